# Initial kernel scaffold; baseline (speedup 1.0000x reference)
import jax
import jax.numpy as jnp
from jax.experimental import pallas as pl


def kernel(points, features, W1, b1, W2, b2, K1, kb1, K2, kb2, P1, pb1, P2, pb2):
    raise NotImplementedError("write your pallas kernel here")


# speedup vs baseline: 1.1514x; 1.1514x over previous
import functools

import jax
import jax.numpy as jnp
from jax.experimental import pallas as pl
from jax.experimental.pallas import tpu as pltpu

RADIUS = 0.1


def _gelu(x):
    c = jnp.sqrt(2.0 / jnp.pi).astype(x.dtype)
    return 0.5 * x * (1.0 + jnp.tanh(c * (x + 0.044715 * (x * x * x))))


def _prep_body(f_ref, pts_ref, W1_ref, b1_ref, W2_ref, b2_ref,
               K1y_ref, K1x_ref, K1f_ref, kb1_ref,
               fy_ref, u_ref, v_ref):
    f = f_ref[...]
    pts = pts_ref[...]
    h = _gelu(jnp.dot(f, W1_ref[...], preferred_element_type=jnp.float32)
              + b1_ref[...])
    fy = jnp.dot(h, W2_ref[...], preferred_element_type=jnp.float32) + b2_ref[...]
    fy_ref[...] = fy
    u_ref[...] = (jnp.dot(pts, K1y_ref[...], preferred_element_type=jnp.float32)
                  + jnp.dot(fy, K1f_ref[...], preferred_element_type=jnp.float32)
                  + kb1_ref[...])
    v_ref[...] = jnp.dot(pts, K1x_ref[...], preferred_element_type=jnp.float32)


def _pairs_body(ptsq_ref, ptst_ref, u_ref, fy_ref, v_ref,
                K2_ref, kb2_ref, P1_ref, pb1_ref, P2_ref, pb2_ref,
                out_ref, *, n_pad, tile_i, tile_j):
    r2 = jnp.float32(RADIUS * RADIUS)
    xq = ptsq_ref[...]
    v_i = v_ref[...]
    K2b = K2_ref[...].astype(jnp.bfloat16)

    n_chunks = n_pad // tile_j

    def body(c, carry):
        s, cnt = carry
        j0 = c * tile_j
        yx = ptst_ref[0:1, pl.ds(j0, tile_j)]
        yy = ptst_ref[1:2, pl.ds(j0, tile_j)]
        yz = ptst_ref[2:3, pl.ds(j0, tile_j)]
        dx = xq[:, 0:1] - yx
        dy = xq[:, 1:2] - yy
        dz = xq[:, 2:3] - yz
        d2 = dx * dx + dy * dy + dz * dz
        mask = (d2 <= r2).astype(jnp.float32)
        u_c = u_ref[pl.ds(j0, tile_j), :]
        fy_c = fy_ref[pl.ds(j0, tile_j), :]
        hidden = _gelu(u_c[None, :, :] + v_i[:, None, :])
        hb = hidden.astype(jnp.bfloat16).reshape(tile_i * tile_j, -1)
        kker = jnp.dot(hb, K2b, preferred_element_type=jnp.float32)
        kker = kker.reshape(tile_i, tile_j, -1) + kb2_ref[...][None, :, :]
        msg = kker * fy_c[None, :, :]
        s = s + jnp.sum(msg * mask[:, :, None], axis=1)
        cnt = cnt + jnp.sum(mask, axis=1, keepdims=True)
        return s, cnt

    H = v_i.shape[-1]
    s0 = jnp.zeros((tile_i, H), jnp.float32)
    c0 = jnp.zeros((tile_i, 1), jnp.float32)
    s, cnt = jax.lax.fori_loop(0, n_chunks, body, (s0, c0))
    h = s / jnp.maximum(cnt, 1.0)
    o = _gelu(jnp.dot(h, P1_ref[...], preferred_element_type=jnp.float32)
              + pb1_ref[...])
    out_ref[...] = (jnp.dot(o, P2_ref[...], preferred_element_type=jnp.float32)
                    + pb2_ref[...])


def kernel(points, features, W1, b1, W2, b2, K1, kb1, K2, kb2, P1, pb1, P2, pb2):
    B, N, _ = points.shape
    IN_C = features.shape[-1]
    H = K2.shape[0]
    OUT_C = P2.shape[1]

    TILE_I = 64
    TILE_J = 256
    n_pad = ((N + TILE_J - 1) // TILE_J) * TILE_J

    outs = []
    for b in range(B):
        pts = points[b]
        f = features[b]

        prep = pl.pallas_call(
            _prep_body,
            out_shape=[
                jax.ShapeDtypeStruct((N, H), jnp.float32),
                jax.ShapeDtypeStruct((N, H), jnp.float32),
                jax.ShapeDtypeStruct((N, H), jnp.float32),
            ],
        )
        fy, u, v = prep(
            f, jnp.pad(pts, ((0, 0), (0, 1))),
            W1, b1[None, :], W2, b2[None, :],
            jnp.pad(K1[0:3], ((0, 1), (0, 0))),
            jnp.pad(K1[3:6], ((0, 1), (0, 0))),
            K1[6:], kb1[None, :],
        )

        pad = n_pad - N
        ptsq = jnp.pad(jnp.pad(pts, ((0, 0), (0, 1))), ((0, pad), (0, 0)),
                       constant_values=1e6)
        ptst = jnp.pad(pts.T, ((0, 5), (0, pad)), constant_values=1e6)
        u_p = jnp.pad(u, ((0, pad), (0, 0)))
        fy_p = jnp.pad(fy, ((0, pad), (0, 0)))
        v_p = jnp.pad(v, ((0, pad), (0, 0)))

        grid = (n_pad // TILE_I,)
        pairs = pl.pallas_call(
            functools.partial(_pairs_body, n_pad=n_pad,
                              tile_i=TILE_I, tile_j=TILE_J),
            grid=grid,
            in_specs=[
                pl.BlockSpec((TILE_I, 4), lambda i: (i, 0)),
                pl.BlockSpec((8, n_pad), lambda i: (0, 0)),
                pl.BlockSpec((n_pad, H), lambda i: (0, 0)),
                pl.BlockSpec((n_pad, H), lambda i: (0, 0)),
                pl.BlockSpec((TILE_I, H), lambda i: (i, 0)),
                pl.BlockSpec((H, H), lambda i: (0, 0)),
                pl.BlockSpec((1, H), lambda i: (0, 0)),
                pl.BlockSpec((H, H), lambda i: (0, 0)),
                pl.BlockSpec((1, H), lambda i: (0, 0)),
                pl.BlockSpec((H, OUT_C), lambda i: (0, 0)),
                pl.BlockSpec((1, OUT_C), lambda i: (0, 0)),
            ],
            out_specs=pl.BlockSpec((TILE_I, OUT_C), lambda i: (i, 0)),
            out_shape=jax.ShapeDtypeStruct((n_pad, OUT_C), jnp.float32),
        )
        o = pairs(ptsq, ptst, u_p, fy_p, v_p,
                  K2, kb2[None, :], P1, pb1[None, :], P2, pb2[None, :])
        outs.append(o[:N])
    return jnp.stack(outs, axis=0)
